# Initial kernel scaffold; baseline (speedup 1.0000x reference)
import jax
import jax.numpy as jnp
from jax.experimental import pallas as pl


def kernel(feat0, feat1, feat2, logits, labels, pos, W0, b0, W1, b1, W2, b2, BE1, be1, BE2, be2, A1, a1, A2, a2, O1, o1, O2, o2):
    raise NotImplementedError("write your pallas kernel here")


# speedup vs baseline: 15.7753x; 15.7753x over previous
import dataclasses
import functools

import jax
import jax.numpy as jnp
import numpy as np
from jax.experimental import pallas as pl
from jax.experimental.pallas import tpu as pltpu
from jax.experimental.pallas import tpu_sc as plsc

TEMP = 0.75
BIG = 3.0e38
QA = 512
QC = 512


def _knn_body(kpos_ref, qpos_ref, dist_ref, idx_ref, d2_ref, *, n_keys, k_nn):
    b = pl.program_id(0)
    q = pl.program_id(1)
    kp = kpos_ref[0]
    qp = qpos_ref[0]
    d2 = ((qp[:, 0:1] - kp[0:1, :]) ** 2
          + (qp[:, 1:2] - kp[1:2, :]) ** 2
          + (qp[:, 2:3] - kp[2:3, :]) ** 2)
    cols = jax.lax.broadcasted_iota(jnp.int32, (QA, n_keys), 1)
    rows = jax.lax.broadcasted_iota(jnp.int32, (QA, n_keys), 0) + q * QA
    d2_ref[...] = jnp.where(cols == rows, BIG, d2)
    base = b * n_keys
    for k in range(k_nn):
        d2v = d2_ref[...]
        m = jnp.min(d2v, axis=1, keepdims=True)
        jmin = jnp.argmin(d2v, axis=1, keepdims=True).astype(jnp.int32)
        dist_ref[0, :, k:k + 1] = jnp.sqrt(m)
        idx_ref[0, :, k:k + 1] = jmin + base
        d2_ref[...] = jnp.where(cols == jmin, BIG, d2v)


def _knn_pairs(kpos, qpos, k_nn):
    bsz, _, n = kpos.shape
    body = functools.partial(_knn_body, n_keys=n, k_nn=k_nn)
    return pl.pallas_call(
        body,
        grid=(bsz, n // QA),
        in_specs=[
            pl.BlockSpec((1, 3, n), lambda b, q: (b, 0, 0)),
            pl.BlockSpec((1, QA, 3), lambda b, q: (b, q, 0)),
        ],
        out_specs=[
            pl.BlockSpec((1, QA, k_nn), lambda b, q: (b, q, 0)),
            pl.BlockSpec((1, QA, k_nn), lambda b, q: (b, q, 0)),
        ],
        out_shape=[
            jax.ShapeDtypeStruct((bsz, n, k_nn), jnp.float32),
            jax.ShapeDtypeStruct((bsz, n, k_nn), jnp.int32),
        ],
        scratch_shapes=[pltpu.VMEM((QA, n), jnp.float32)],
        compiler_params=pltpu.CompilerParams(
            dimension_semantics=("parallel", "parallel")),
    )(kpos, qpos)


def _gather_labels(labels_flat, idx_flat):
    m = idx_flat.shape[0]
    nv = labels_flat.shape[0]
    n_workers = 32
    lanes = 16
    b_per_w = m // n_workers
    mesh = plsc.VectorSubcoreMesh(core_axis_name="c", subcore_axis_name="s")
    cp = pltpu.CompilerParams()
    if "needs_layout_passes" in pltpu.CompilerParams.__dataclass_fields__:
        cp = dataclasses.replace(cp, needs_layout_passes=False)

    @functools.partial(
        pl.kernel,
        out_type=jax.ShapeDtypeStruct((m,), jnp.int32),
        mesh=mesh,
        compiler_params=cp,
        scratch_types=[
            pltpu.VMEM((nv,), jnp.int32),
            pltpu.VMEM((b_per_w,), jnp.int32),
            pltpu.VMEM((b_per_w,), jnp.int32),
            pltpu.SemaphoreType.DMA,
        ],
    )
    def gather_kernel(lab_hbm, idx_hbm, out_hbm, lab_v, idx_v, out_v, sem):
        wid = jax.lax.axis_index("s") * 2 + jax.lax.axis_index("c")
        base = wid * b_per_w
        pltpu.async_copy(lab_hbm, lab_v, sem).wait()
        pltpu.async_copy(idx_hbm.at[pl.ds(base, b_per_w)], idx_v, sem).wait()

        @pl.loop(0, b_per_w, step=lanes)
        def _(j):
            iv = idx_v[pl.ds(j, lanes)]
            out_v[pl.ds(j, lanes)] = plsc.load_gather(lab_v, [iv])

        pltpu.async_copy(out_v, out_hbm.at[pl.ds(base, b_per_w)], sem).wait()

    return gather_kernel(labels_flat, idx_flat)


def _fuse_body(f0_ref, f1_ref, f2_ref, lg_ref, lab_ref, dist_ref, nlab_ref,
               W0_ref, b0_ref, W1_ref, b1_ref, W2_ref, b2_ref,
               BE1_ref, be1_ref, BE2_ref, be2_ref,
               A1_ref, a1_ref, A2_ref, a2_ref,
               O1_ref, o1_ref, O2_ref, o2_ref,
               out_ref, attn_ref, *, k_nn, n_cls, rd):
    dot = functools.partial(jnp.dot, preferred_element_type=jnp.float32)
    relu = lambda x: jnp.maximum(x, 0.0)

    dist = dist_ref[0]
    nlab = nlab_ref[0]
    lab = lab_ref[0]
    diff = (nlab != lab).astype(jnp.float32)
    kf = float(k_nn)
    dr = jnp.sum(diff, axis=1, keepdims=True) / kf
    same = 1.0 - diff
    same_dist = (jnp.sum(dist * same, axis=1, keepdims=True)
                 / (jnp.sum(same, axis=1, keepdims=True) + 1e-6))
    bmin = jnp.min(jnp.where(diff > 0.0, dist, BIG), axis=1, keepdims=True)
    bdist = jnp.where(bmin < 1e30, bmin, same_dist)
    dmean = jnp.sum(dist, axis=1, keepdims=True) / kf
    density = 1.0 / (dmean + 1e-6)
    var = jnp.sum((dist - dmean) ** 2, axis=1, keepdims=True) / (kf - 1.0)
    curvature = jnp.sqrt(var) / (dmean + 1e-6)

    lg = lg_ref[0] / TEMP
    lmax = jnp.max(lg, axis=1, keepdims=True)
    e = jnp.exp(lg - lmax)
    s = jnp.sum(e, axis=1, keepdims=True)
    probs = e / s
    conf = jnp.max(probs, axis=1, keepdims=True)
    ent = -jnp.sum(probs * jnp.log(probs + 1e-8), axis=1, keepdims=True) \
        * (1.0 / float(np.log(n_cls)))

    h1 = (be1_ref[...]
          + dr * BE1_ref[0:1, :] + conf * BE1_ref[1:2, :]
          + ent * BE1_ref[2:3, :] + density * BE1_ref[3:4, :]
          + curvature * BE1_ref[4:5, :] + bdist * BE1_ref[5:6, :])
    benc = relu(dot(relu(h1), BE2_ref[...]) + be2_ref[...])

    f0 = dot(f0_ref[0], W0_ref[...]) + b0_ref[...]
    f1 = dot(f1_ref[0], W1_ref[...]) + b1_ref[...]
    f2 = dot(f2_ref[0], W2_ref[...]) + b2_ref[...]
    gf = (f0 + f1 + f2) * (1.0 / 3.0)

    h = relu(dot(gf, A1_ref[0:rd, :]) + dot(benc, A1_ref[rd:, :])
             + a1_ref[...])
    al = dot(h, A2_ref[...]) + a2_ref[...]
    amax = jnp.max(al, axis=1, keepdims=True)
    ae = jnp.exp(al - amax)
    attn = ae / jnp.sum(ae, axis=1, keepdims=True)
    attn_ref[0] = attn

    fused = f0 * attn[:, 0:1] + f1 * attn[:, 1:2] + f2 * attn[:, 2:3]
    out_ref[0] = (dot(relu(dot(fused, O1_ref[...]) + o1_ref[...]),
                      O2_ref[...]) + o2_ref[...] + gf)


def _fuse(feat0, feat1, feat2, logits, labels3, dist, nlab,
          W0, b0, W1, b1, W2, b2, BE1, be1, BE2, be2,
          A1, a1, A2, a2, O1, o1, O2, o2):
    bsz, n, _ = feat0.shape
    k_nn = dist.shape[2]
    n_cls = logits.shape[2]
    rd = W0.shape[1]
    body = functools.partial(_fuse_body, k_nn=k_nn, n_cls=n_cls, rd=rd)
    row = lambda b, q: (b, q, 0)
    full = lambda b, q: (0, 0)

    def wspec(w):
        return pl.BlockSpec(w.shape, full)

    return pl.pallas_call(
        body,
        grid=(bsz, n // QC),
        in_specs=[
            pl.BlockSpec((1, QC, feat0.shape[2]), row),
            pl.BlockSpec((1, QC, feat1.shape[2]), row),
            pl.BlockSpec((1, QC, feat2.shape[2]), row),
            pl.BlockSpec((1, QC, n_cls), row),
            pl.BlockSpec((1, QC, 1), row),
            pl.BlockSpec((1, QC, k_nn), row),
            pl.BlockSpec((1, QC, k_nn), row),
            wspec(W0), wspec(b0), wspec(W1), wspec(b1), wspec(W2), wspec(b2),
            wspec(BE1), wspec(be1), wspec(BE2), wspec(be2),
            wspec(A1), wspec(a1), wspec(A2), wspec(a2),
            wspec(O1), wspec(o1), wspec(O2), wspec(o2),
        ],
        out_specs=[
            pl.BlockSpec((1, QC, rd), row),
            pl.BlockSpec((1, QC, 3), row),
        ],
        out_shape=[
            jax.ShapeDtypeStruct((bsz, n, rd), jnp.float32),
            jax.ShapeDtypeStruct((bsz, n, 3), jnp.float32),
        ],
        compiler_params=pltpu.CompilerParams(
            dimension_semantics=("parallel", "parallel")),
    )(feat0, feat1, feat2, logits, labels3, dist, nlab,
      W0, b0, W1, b1, W2, b2, BE1, be1, BE2, be2,
      A1, a1, A2, a2, O1, o1, O2, o2)


def kernel(feat0, feat1, feat2, logits, labels, pos,
           W0, b0, W1, b1, W2, b2,
           BE1, be1, BE2, be2,
           A1, a1, A2, a2,
           O1, o1, O2, o2):
    bsz, n, _ = pos.shape
    k_nn = 12
    kpos = jnp.transpose(pos, (0, 2, 1))
    dist, idx = _knn_pairs(kpos, pos, k_nn)
    nlab_flat = _gather_labels(labels.reshape(bsz * n),
                               idx.reshape(bsz * n * k_nn))
    nlab = nlab_flat.reshape(bsz, n, k_nn)
    out, attn = _fuse(
        feat0, feat1, feat2, logits, labels.reshape(bsz, n, 1), dist, nlab,
        W0, b0.reshape(1, -1), W1, b1.reshape(1, -1), W2, b2.reshape(1, -1),
        BE1, be1.reshape(1, -1), BE2, be2.reshape(1, -1),
        A1, a1.reshape(1, -1), A2, a2.reshape(1, -1),
        O1, o1.reshape(1, -1), O2, o2.reshape(1, -1))
    return out, attn
